# Initial kernel scaffold; baseline (speedup 1.0000x reference)
import jax
import jax.numpy as jnp
from jax.experimental import pallas as pl


def kernel(x, adj, weight, bias):
    raise NotImplementedError("write your pallas kernel here")


# speedup vs baseline: 2.2163x; 2.2163x over previous
import jax
import jax.numpy as jnp
from jax.experimental import pallas as pl
from jax.experimental.pallas import tpu as pltpu


def _fused_gcn_kernel(adj_ref, x_ref, wbd_ref, b_ref, o_ref):
    a = adj_ref[...].astype(jnp.bfloat16)
    t = jnp.dot(a, x_ref[...], preferred_element_type=jnp.float32)
    out = jnp.dot(t.astype(jnp.bfloat16), wbd_ref[...],
                  preferred_element_type=jnp.float32)
    o_ref[...] = out + b_ref[...]


def kernel(x, adj, weight, bias):
    N, S, F = x.shape
    cols = S * F

    x_flat = x.reshape(N, cols).astype(jnp.bfloat16)
    wbd = jnp.kron(jnp.eye(S, dtype=weight.dtype), weight).astype(jnp.bfloat16)
    b_row = jnp.tile(bias, (S,)).reshape(1, cols).astype(jnp.float32)

    bm = 256 if N % 256 == 0 else N

    out_flat = pl.pallas_call(
        _fused_gcn_kernel,
        out_shape=jax.ShapeDtypeStruct((N, cols), x.dtype),
        grid=(N // bm,),
        in_specs=[
            pl.BlockSpec((bm, N), lambda i: (i, 0)),
            pl.BlockSpec((N, cols), lambda i: (0, 0)),
            pl.BlockSpec((cols, cols), lambda i: (0, 0)),
            pl.BlockSpec((1, cols), lambda i: (0, 0)),
        ],
        out_specs=pl.BlockSpec((bm, cols), lambda i: (i, 0)),
        compiler_params=pltpu.CompilerParams(
            dimension_semantics=("parallel",)),
    )(adj, x_flat, wbd, b_row)

    return out_flat.reshape(N, S, F)
